# Initial kernel scaffold; baseline (speedup 1.0000x reference)
import jax
import jax.numpy as jnp
from jax.experimental import pallas as pl


def kernel(x0, edge_index0, batch0, x1, edge_index1, batch1, W1, b1, W2, b2, Wc1, bc1, Wc2, bc2):
    raise NotImplementedError("write your pallas kernel here")


# speedup vs baseline: 9.9488x; 9.9488x over previous
import functools

import jax
import jax.numpy as jnp
from jax import lax
from jax.experimental import pallas as pl
from jax.experimental.pallas import tpu as pltpu
from jax.experimental.pallas import tpu_sc as plsc

N = 10000
NP = 10240
E = 320000
D = 128
B = 16

NT = 16
EPT = E // NT
NPT = NP // NT
K = 80
NCHUNK = EPT // K

_mesh = plsc.VectorSubcoreMesh(core_axis_name="c", subcore_axis_name="s")


@functools.partial(
    pl.kernel,
    out_type=(
        jax.ShapeDtypeStruct((NP, D), jnp.float32),
        jax.ShapeDtypeStruct((NP, D), jnp.float32),
    ),
    mesh=_mesh,
    scratch_types=[
        pltpu.VMEM((K,), jnp.int32),
        pltpu.VMEM((K, D), jnp.float32),
        pltpu.VMEM_SHARED((NP, D), jnp.float32),
    ],
)
def _deg_kernel(dst0, dst1, ones_hbm, zeros_hbm, deg0, deg1,
                idx_v, ones_v, acc_sh):
    c = lax.axis_index("c")
    s = lax.axis_index("s")
    pltpu.sync_copy(zeros_hbm, acc_sh.at[pl.ds(s * NPT, NPT)])
    pltpu.sync_copy(ones_hbm, ones_v)
    plsc.subcore_barrier()

    def run(dst_hbm, out_hbm):
        def body(i, carry):
            pltpu.sync_copy(dst_hbm.at[pl.ds(s * EPT + i * K, K)], idx_v)
            pltpu.sync_copy(ones_v, acc_sh.at[idx_v], add=True)
            return carry
        lax.fori_loop(0, NCHUNK, body, 0)
        plsc.subcore_barrier()
        pltpu.sync_copy(acc_sh.at[pl.ds(s * NPT, NPT)],
                        out_hbm.at[pl.ds(s * NPT, NPT)])

    @pl.when(c == 0)
    def _():
        run(dst0, deg0)

    @pl.when(c == 1)
    def _():
        run(dst1, deg1)


@functools.partial(
    pl.kernel,
    out_type=(
        jax.ShapeDtypeStruct((NP, D), jnp.float32),
        jax.ShapeDtypeStruct((NP, D), jnp.float32),
    ),
    mesh=_mesh,
    scratch_types=[
        pltpu.VMEM((K,), jnp.int32),
        pltpu.VMEM((K,), jnp.int32),
        pltpu.VMEM((K, D), jnp.float32),
        pltpu.SemaphoreType.DMA,
        pltpu.VMEM_SHARED((NP, D), jnp.float32),
    ],
)
def _scatter_kernel(src0, dst0, src1, dst1, y0, y1, zeros_hbm, s0_out, s1_out,
                    sidx_v, didx_v, rows_v, sem, acc_sh):
    c = lax.axis_index("c")
    s = lax.axis_index("s")
    pltpu.sync_copy(zeros_hbm, acc_sh.at[pl.ds(s * NPT, NPT)])
    plsc.subcore_barrier()

    def run(src_hbm, dst_hbm, y_hbm, out_hbm):
        def body(i, carry):
            e0 = s * EPT + i * K
            pltpu.sync_copy(src_hbm.at[pl.ds(e0, K)], sidx_v)
            pltpu.sync_copy(dst_hbm.at[pl.ds(e0, K)], didx_v)
            pltpu.async_copy(y_hbm.at[sidx_v], rows_v, sem).wait()
            pltpu.sync_copy(rows_v, acc_sh.at[didx_v], add=True)
            return carry
        lax.fori_loop(0, NCHUNK, body, 0)
        plsc.subcore_barrier()
        pltpu.sync_copy(acc_sh.at[pl.ds(s * NPT, NPT)],
                        out_hbm.at[pl.ds(s * NPT, NPT)])

    @pl.when(c == 0)
    def _():
        run(src0, dst0, y0, s0_out)

    @pl.when(c == 1)
    def _():
        run(src1, dst1, y1, s1_out)


def _mm1_body(x_ref, deg_ref, w_ref, y_ref):
    dinv = lax.rsqrt(deg_ref[0][:, 0:1] + 1.0)
    y_ref[0] = dinv * jnp.dot(x_ref[0], w_ref[...],
                              preferred_element_type=jnp.float32)


NB = 4
RB = NP // NB


def _mm1(xs, deg, w1):
    return pl.pallas_call(
        _mm1_body,
        grid=(2, NB),
        in_specs=[
            pl.BlockSpec((1, RB, D), lambda g, r: (g, r, 0)),
            pl.BlockSpec((1, RB, D), lambda g, r: (g, r, 0)),
            pl.BlockSpec((D, D), lambda g, r: (0, 0)),
        ],
        out_specs=pl.BlockSpec((1, RB, D), lambda g, r: (g, r, 0)),
        out_shape=jax.ShapeDtypeStruct((2, NP, D), jnp.float32),
    )(xs, deg, w1)


def _mm2_body(s_ref, y_ref, deg_ref, w_ref, b_ref, out_ref):
    dinv = lax.rsqrt(deg_ref[0][:, 0:1] + 1.0)
    h = dinv * (s_ref[0] + y_ref[0]) + b_ref[...]
    out_ref[0] = dinv * jnp.dot(h, w_ref[...],
                                preferred_element_type=jnp.float32)


def _mm2(s1, y1, deg, w2, b1):
    return pl.pallas_call(
        _mm2_body,
        grid=(2, NB),
        in_specs=[
            pl.BlockSpec((1, RB, D), lambda g, r: (g, r, 0)),
            pl.BlockSpec((1, RB, D), lambda g, r: (g, r, 0)),
            pl.BlockSpec((1, RB, D), lambda g, r: (g, r, 0)),
            pl.BlockSpec((D, D), lambda g, r: (0, 0)),
            pl.BlockSpec((1, D), lambda g, r: (0, 0)),
        ],
        out_specs=pl.BlockSpec((1, RB, D), lambda g, r: (g, r, 0)),
        out_shape=jax.ShapeDtypeStruct((2, NP, D), jnp.float32),
    )(s1, y1, deg, w2, b1)


def _final_body(s_ref, y_ref, deg_ref, batch_ref, b2_ref, wc1_ref, bc1_ref,
                wc2_ref, bc2_ref, out_ref):
    feats = []
    for g in range(2):
        dinv = lax.rsqrt(deg_ref[g][:, 0:1] + 1.0)
        h2 = dinv * (s_ref[g] + y_ref[g]) + b2_ref[...]
        iota = lax.broadcasted_iota(jnp.int32, (NP, 16), 1)
        mask = (batch_ref[g] == iota).astype(jnp.float32)
        cnt = jnp.maximum(jnp.sum(mask, axis=0, keepdims=True), 1.0)
        meanmask = mask / cnt
        feats.append(lax.dot_general(
            meanmask, h2, (((0,), (0,)), ((), ())),
            preferred_element_type=jnp.float32))
    cf = jnp.concatenate(feats, axis=1)
    h = jax.nn.sigmoid(jnp.dot(cf, wc1_ref[...],
                               preferred_element_type=jnp.float32)
                       + bc1_ref[...])
    logit = jnp.dot(h, wc2_ref[...],
                    preferred_element_type=jnp.float32) + bc2_ref[...]
    out_ref[...] = jax.nn.sigmoid(logit)


def _final(s2, y2, deg, batch, b2, wc1, bc1, wc2p, bc2r):
    return pl.pallas_call(
        _final_body,
        out_shape=jax.ShapeDtypeStruct((B, D), jnp.float32),
    )(s2, y2, deg, batch, b2, wc1, bc1, wc2p, bc2r)


def kernel(x0, edge_index0, batch0, x1, edge_index1, batch1,
           W1, b1, W2, b2, Wc1, bc1, Wc2, bc2):
    src0, dst0 = edge_index0[0], edge_index0[1]
    src1, dst1 = edge_index1[0], edge_index1[1]
    pad = NP - N
    xs = jnp.pad(jnp.stack([x0, x1]), ((0, 0), (0, pad), (0, 0)))

    ones128 = jnp.ones((K, D), jnp.float32)
    zeros128 = jnp.zeros((NPT, D), jnp.float32)

    deg0, deg1 = _deg_kernel(dst0, dst1, ones128, zeros128)
    deg = jnp.stack([deg0, deg1])

    y1 = _mm1(xs, deg, W1)
    s1a, s1b = _scatter_kernel(src0, dst0, src1, dst1, y1[0], y1[1], zeros128)
    s1 = jnp.stack([s1a, s1b])

    y2 = _mm2(s1, y1, deg, W2, b1.reshape(1, D))
    s2a, s2b = _scatter_kernel(src0, dst0, src1, dst1, y2[0], y2[1], zeros128)
    s2 = jnp.stack([s2a, s2b])

    batch = jnp.broadcast_to(
        jnp.pad(jnp.stack([batch0, batch1]), ((0, 0), (0, pad)),
                constant_values=B)[:, :, None], (2, NP, 16))
    wc2p = jnp.pad(Wc2, ((0, 0), (0, D - 1)))
    bc2r = jnp.broadcast_to(bc2[None, :], (1, D))
    out = _final(s2, y2, deg, batch, b2.reshape(1, D), Wc1,
                 bc1.reshape(1, D), wc2p, bc2r)
    return out[:, 0]


# speedup vs baseline: 14.0925x; 1.4165x over previous
import functools

import jax
import jax.numpy as jnp
from jax import lax
from jax.experimental import pallas as pl
from jax.experimental.pallas import tpu as pltpu
from jax.experimental.pallas import tpu_sc as plsc

N = 10000
NP = 10240
E = 320000
D = 128
B = 16

NT = 16
EPT = E // NT
NPT = NP // NT
K = 80
NCHUNK = EPT // K
UNROLL = 2
NITER = NCHUNK // UNROLL

_mesh = plsc.VectorSubcoreMesh(core_axis_name="c", subcore_axis_name="s")


@functools.partial(
    pl.kernel,
    out_type=(
        jax.ShapeDtypeStruct((NP, D), jnp.float32),
        jax.ShapeDtypeStruct((NP, D), jnp.float32),
    ),
    mesh=_mesh,
    scratch_types=[
        pltpu.VMEM((UNROLL, K), jnp.int32),
        pltpu.VMEM((K, D), jnp.float32),
        pltpu.SemaphoreType.DMA,
        pltpu.VMEM_SHARED((NP, D), jnp.float32),
    ],
)
def _deg_kernel(dst0, dst1, ones_hbm, zeros_hbm, deg0, deg1,
                didx_v, ones_v, ssem, acc_sh):
    c = lax.axis_index("c")
    s = lax.axis_index("s")
    pltpu.sync_copy(zeros_hbm, acc_sh.at[pl.ds(s * NPT, NPT)])
    pltpu.sync_copy(ones_hbm, ones_v)
    plsc.subcore_barrier()

    def run(dst_hbm, out_hbm):
        def body(i, carry):
            pltpu.sync_copy(dst_hbm.at[s, i], didx_v)
            descs = [
                pltpu.async_copy(ones_v, acc_sh.at[didx_v.at[j]],
                                 ssem, add=True)
                for j in range(UNROLL)
            ]
            for d in descs:
                d.wait()
            return carry
        lax.fori_loop(0, NITER, body, 0)
        plsc.subcore_barrier()
        pltpu.sync_copy(acc_sh.at[pl.ds(s * NPT, NPT)],
                        out_hbm.at[pl.ds(s * NPT, NPT)])

    @pl.when(c == 0)
    def _():
        run(dst0, deg0)

    @pl.when(c == 1)
    def _():
        run(dst1, deg1)


@functools.partial(
    pl.kernel,
    out_type=(
        jax.ShapeDtypeStruct((NP, D), jnp.float32),
        jax.ShapeDtypeStruct((NP, D), jnp.float32),
    ),
    mesh=_mesh,
    scratch_types=[
        pltpu.VMEM((UNROLL, K), jnp.int32),
        pltpu.VMEM((UNROLL, K), jnp.int32),
        [pltpu.VMEM((K, D), jnp.float32) for _ in range(UNROLL)],
        pltpu.SemaphoreType.DMA,
        pltpu.SemaphoreType.DMA,
        pltpu.VMEM_SHARED((NP, D), jnp.float32),
    ],
)
def _scatter_kernel(src0, dst0, src1, dst1, y0, y1, zeros_hbm, s0_out, s1_out,
                    sidx_v, didx_v, rows_v, gsem, ssem, acc_sh):
    c = lax.axis_index("c")
    s = lax.axis_index("s")
    pltpu.sync_copy(zeros_hbm, acc_sh.at[pl.ds(s * NPT, NPT)])
    plsc.subcore_barrier()

    def run(src_hbm, dst_hbm, y_hbm, out_hbm):
        def body(i, carry):
            pltpu.sync_copy(src_hbm.at[s, i], sidx_v)
            pltpu.sync_copy(dst_hbm.at[s, i], didx_v)
            gds = [
                pltpu.async_copy(y_hbm.at[sidx_v.at[j]], rows_v[j], gsem)
                for j in range(UNROLL)
            ]
            sds = []
            for j in range(UNROLL):
                gds[j].wait()
                sds.append(pltpu.async_copy(
                    rows_v[j], acc_sh.at[didx_v.at[j]], ssem, add=True))
            for d in sds:
                d.wait()
            return carry
        lax.fori_loop(0, NITER, body, 0)
        plsc.subcore_barrier()
        pltpu.sync_copy(acc_sh.at[pl.ds(s * NPT, NPT)],
                        out_hbm.at[pl.ds(s * NPT, NPT)])

    @pl.when(c == 0)
    def _():
        run(src0, dst0, y0, s0_out)

    @pl.when(c == 1)
    def _():
        run(src1, dst1, y1, s1_out)


def _mm1_body(x_ref, deg_ref, w_ref, y_ref):
    dinv = lax.rsqrt(deg_ref[0][:, 0:1] + 1.0)
    y_ref[0] = dinv * jnp.dot(x_ref[0], w_ref[...],
                              preferred_element_type=jnp.float32)


NB = 4
RB = NP // NB


def _mm1(xs, deg, w1):
    return pl.pallas_call(
        _mm1_body,
        grid=(2, NB),
        in_specs=[
            pl.BlockSpec((1, RB, D), lambda g, r: (g, r, 0)),
            pl.BlockSpec((1, RB, D), lambda g, r: (g, r, 0)),
            pl.BlockSpec((D, D), lambda g, r: (0, 0)),
        ],
        out_specs=pl.BlockSpec((1, RB, D), lambda g, r: (g, r, 0)),
        out_shape=jax.ShapeDtypeStruct((2, NP, D), jnp.float32),
    )(xs, deg, w1)


def _mm2_body(s_ref, y_ref, deg_ref, w_ref, b_ref, out_ref):
    dinv = lax.rsqrt(deg_ref[0][:, 0:1] + 1.0)
    h = dinv * (s_ref[0] + y_ref[0]) + b_ref[...]
    out_ref[0] = dinv * jnp.dot(h, w_ref[...],
                                preferred_element_type=jnp.float32)


def _mm2(s1, y1, deg, w2, b1):
    return pl.pallas_call(
        _mm2_body,
        grid=(2, NB),
        in_specs=[
            pl.BlockSpec((1, RB, D), lambda g, r: (g, r, 0)),
            pl.BlockSpec((1, RB, D), lambda g, r: (g, r, 0)),
            pl.BlockSpec((1, RB, D), lambda g, r: (g, r, 0)),
            pl.BlockSpec((D, D), lambda g, r: (0, 0)),
            pl.BlockSpec((1, D), lambda g, r: (0, 0)),
        ],
        out_specs=pl.BlockSpec((1, RB, D), lambda g, r: (g, r, 0)),
        out_shape=jax.ShapeDtypeStruct((2, NP, D), jnp.float32),
    )(s1, y1, deg, w2, b1)


def _final_body(s_ref, y_ref, deg_ref, batch_ref, b2_ref, wc1_ref, bc1_ref,
                wc2_ref, bc2_ref, out_ref):
    feats = []
    for g in range(2):
        dinv = lax.rsqrt(deg_ref[g][:, 0:1] + 1.0)
        h2 = dinv * (s_ref[g] + y_ref[g]) + b2_ref[...]
        iota = lax.broadcasted_iota(jnp.int32, (NP, 16), 1)
        mask = (batch_ref[g] == iota).astype(jnp.float32)
        cnt = jnp.maximum(jnp.sum(mask, axis=0, keepdims=True), 1.0)
        meanmask = mask / cnt
        feats.append(lax.dot_general(
            meanmask, h2, (((0,), (0,)), ((), ())),
            preferred_element_type=jnp.float32))
    cf = jnp.concatenate(feats, axis=1)
    h = jax.nn.sigmoid(jnp.dot(cf, wc1_ref[...],
                               preferred_element_type=jnp.float32)
                       + bc1_ref[...])
    logit = jnp.dot(h, wc2_ref[...],
                    preferred_element_type=jnp.float32) + bc2_ref[...]
    out_ref[...] = jax.nn.sigmoid(logit)


def _final(s2, y2, deg, batch, b2, wc1, bc1, wc2p, bc2r):
    return pl.pallas_call(
        _final_body,
        out_shape=jax.ShapeDtypeStruct((B, D), jnp.float32),
    )(s2, y2, deg, batch, b2, wc1, bc1, wc2p, bc2r)


def kernel(x0, edge_index0, batch0, x1, edge_index1, batch1,
           W1, b1, W2, b2, Wc1, bc1, Wc2, bc2):
    src0 = edge_index0[0].reshape(NT, NITER, UNROLL, K)
    dst0 = edge_index0[1].reshape(NT, NITER, UNROLL, K)
    src1 = edge_index1[0].reshape(NT, NITER, UNROLL, K)
    dst1 = edge_index1[1].reshape(NT, NITER, UNROLL, K)
    pad = NP - N
    xs = jnp.pad(jnp.stack([x0, x1]), ((0, 0), (0, pad), (0, 0)))

    ones128 = jnp.ones((K, D), jnp.float32)
    zeros128 = jnp.zeros((NPT, D), jnp.float32)

    deg0, deg1 = _deg_kernel(dst0, dst1, ones128, zeros128)
    deg = jnp.stack([deg0, deg1])

    y1 = _mm1(xs, deg, W1)
    s1a, s1b = _scatter_kernel(src0, dst0, src1, dst1, y1[0], y1[1], zeros128)
    s1 = jnp.stack([s1a, s1b])

    y2 = _mm2(s1, y1, deg, W2, b1.reshape(1, D))
    s2a, s2b = _scatter_kernel(src0, dst0, src1, dst1, y2[0], y2[1], zeros128)
    s2 = jnp.stack([s2a, s2b])

    batch = jnp.broadcast_to(
        jnp.pad(jnp.stack([batch0, batch1]), ((0, 0), (0, pad)),
                constant_values=B)[:, :, None], (2, NP, 16))
    wc2p = jnp.pad(Wc2, ((0, 0), (0, D - 1)))
    bc2r = jnp.broadcast_to(bc2[None, :], (1, D))
    out = _final(s2, y2, deg, batch, b2.reshape(1, D), Wc1,
                 bc1.reshape(1, D), wc2p, bc2r)
    return out[:, 0]


# speedup vs baseline: 22.1716x; 1.5733x over previous
import functools

import jax
import jax.numpy as jnp
from jax import lax
from jax.experimental import pallas as pl
from jax.experimental.pallas import tpu as pltpu
from jax.experimental.pallas import tpu_sc as plsc

N = 10000
NP = 10240
E = 320000
D = 128
B = 16

NT = 16
EPT = E // NT
NPT = NP // NT
K = 80
NCHUNK = EPT // K
G2 = 10
NGRP = NCHUNK // G2
NBUF = 4

_mesh = plsc.VectorSubcoreMesh(core_axis_name="c", subcore_axis_name="s")


@functools.partial(
    pl.kernel,
    out_type=(
        jax.ShapeDtypeStruct((NP, D), jnp.float32),
        jax.ShapeDtypeStruct((NP, D), jnp.float32),
    ),
    mesh=_mesh,
    scratch_types=[
        pltpu.VMEM((G2, K), jnp.int32),
        pltpu.VMEM((K, D), jnp.float32),
        pltpu.SemaphoreType.DMA,
        pltpu.VMEM_SHARED((NP, D), jnp.float32),
    ],
)
def _deg_kernel(dst0, dst1, ones_hbm, zeros_hbm, deg0, deg1,
                didx_v, ones_v, ssem, acc_sh):
    c = lax.axis_index("c")
    s = lax.axis_index("s")
    pltpu.sync_copy(zeros_hbm, acc_sh.at[pl.ds(s * NPT, NPT)])
    pltpu.sync_copy(ones_hbm, ones_v)
    plsc.subcore_barrier()

    def run(dst_hbm, out_hbm):
        def body(g, carry):
            pltpu.sync_copy(dst_hbm.at[s, g], didx_v)
            descs = [
                pltpu.async_copy(ones_v, acc_sh.at[didx_v.at[t]],
                                 ssem, add=True)
                for t in range(G2)
            ]
            for d in descs:
                d.wait()
            return carry
        lax.fori_loop(0, NGRP, body, 0)
        plsc.subcore_barrier()
        pltpu.sync_copy(acc_sh.at[pl.ds(s * NPT, NPT)],
                        out_hbm.at[pl.ds(s * NPT, NPT)])

    @pl.when(c == 0)
    def _():
        run(dst0, deg0)

    @pl.when(c == 1)
    def _():
        run(dst1, deg1)


@functools.partial(
    pl.kernel,
    out_type=(
        jax.ShapeDtypeStruct((NP, D), jnp.float32),
        jax.ShapeDtypeStruct((NP, D), jnp.float32),
    ),
    mesh=_mesh,
    scratch_types=[
        pltpu.VMEM((2, G2, K), jnp.int32),
        [pltpu.VMEM((K, D), jnp.float32) for _ in range(NBUF)],
        pltpu.SemaphoreType.DMA,
        pltpu.SemaphoreType.DMA,
        pltpu.VMEM_SHARED((NP, D), jnp.float32),
    ],
)
def _scatter_kernel(ei0, ei1, y0, y1, zeros_hbm, s0_out, s1_out,
                    idx_v, rows_v, gsem, ssem, acc_sh):
    c = lax.axis_index("c")
    s = lax.axis_index("s")
    pltpu.sync_copy(zeros_hbm, acc_sh.at[pl.ds(s * NPT, NPT)])
    plsc.subcore_barrier()

    def run(ei_hbm, y_hbm, out_hbm):
        def body(g, carry):
            pltpu.sync_copy(ei_hbm.at[s, g], idx_v)
            gds = [None] * G2
            sds = [None] * G2
            for t in range(G2):
                if t >= NBUF:
                    sds[t - NBUF].wait()
                gds[t] = pltpu.async_copy(
                    y_hbm.at[idx_v.at[0, t]], rows_v[t % NBUF], gsem)
                if t >= NBUF - 1:
                    u = t - (NBUF - 1)
                    gds[u].wait()
                    sds[u] = pltpu.async_copy(
                        rows_v[u % NBUF], acc_sh.at[idx_v.at[1, u]],
                        ssem, add=True)
            for u in range(G2 - NBUF + 1, G2):
                gds[u].wait()
                sds[u] = pltpu.async_copy(
                    rows_v[u % NBUF], acc_sh.at[idx_v.at[1, u]],
                    ssem, add=True)
            for u in range(G2 - NBUF, G2):
                sds[u].wait()
            return carry
        lax.fori_loop(0, NGRP, body, 0)
        plsc.subcore_barrier()
        pltpu.sync_copy(acc_sh.at[pl.ds(s * NPT, NPT)],
                        out_hbm.at[pl.ds(s * NPT, NPT)])

    @pl.when(c == 0)
    def _():
        run(ei0, y0, s0_out)

    @pl.when(c == 1)
    def _():
        run(ei1, y1, s1_out)


def _mm1_body(x_ref, deg_ref, w_ref, y_ref):
    dinv = lax.rsqrt(deg_ref[0][:, 0:1] + 1.0)
    y_ref[0] = dinv * jnp.dot(x_ref[0], w_ref[...],
                              preferred_element_type=jnp.float32)


NB = 4
RB = NP // NB


def _mm1(xs, deg, w1):
    return pl.pallas_call(
        _mm1_body,
        grid=(2, NB),
        in_specs=[
            pl.BlockSpec((1, RB, D), lambda g, r: (g, r, 0)),
            pl.BlockSpec((1, RB, D), lambda g, r: (g, r, 0)),
            pl.BlockSpec((D, D), lambda g, r: (0, 0)),
        ],
        out_specs=pl.BlockSpec((1, RB, D), lambda g, r: (g, r, 0)),
        out_shape=jax.ShapeDtypeStruct((2, NP, D), jnp.float32),
    )(xs, deg, w1)


def _mm2_body(s_ref, y_ref, deg_ref, w_ref, b_ref, out_ref):
    dinv = lax.rsqrt(deg_ref[0][:, 0:1] + 1.0)
    h = dinv * (s_ref[0] + y_ref[0]) + b_ref[...]
    out_ref[0] = dinv * jnp.dot(h, w_ref[...],
                                preferred_element_type=jnp.float32)


def _mm2(s1, y1, deg, w2, b1):
    return pl.pallas_call(
        _mm2_body,
        grid=(2, NB),
        in_specs=[
            pl.BlockSpec((1, RB, D), lambda g, r: (g, r, 0)),
            pl.BlockSpec((1, RB, D), lambda g, r: (g, r, 0)),
            pl.BlockSpec((1, RB, D), lambda g, r: (g, r, 0)),
            pl.BlockSpec((D, D), lambda g, r: (0, 0)),
            pl.BlockSpec((1, D), lambda g, r: (0, 0)),
        ],
        out_specs=pl.BlockSpec((1, RB, D), lambda g, r: (g, r, 0)),
        out_shape=jax.ShapeDtypeStruct((2, NP, D), jnp.float32),
    )(s1, y1, deg, w2, b1)


def _final_body(s_ref, y_ref, deg_ref, batch_ref, b2_ref, wc1_ref, bc1_ref,
                wc2_ref, bc2_ref, out_ref):
    feats = []
    for g in range(2):
        dinv = lax.rsqrt(deg_ref[g][:, 0:1] + 1.0)
        h2 = dinv * (s_ref[g] + y_ref[g]) + b2_ref[...]
        iota = lax.broadcasted_iota(jnp.int32, (NP, 16), 1)
        mask = (batch_ref[g] == iota).astype(jnp.float32)
        cnt = jnp.maximum(jnp.sum(mask, axis=0, keepdims=True), 1.0)
        meanmask = mask / cnt
        feats.append(lax.dot_general(
            meanmask, h2, (((0,), (0,)), ((), ())),
            preferred_element_type=jnp.float32))
    cf = jnp.concatenate(feats, axis=1)
    h = jax.nn.sigmoid(jnp.dot(cf, wc1_ref[...],
                               preferred_element_type=jnp.float32)
                       + bc1_ref[...])
    logit = jnp.dot(h, wc2_ref[...],
                    preferred_element_type=jnp.float32) + bc2_ref[...]
    out_ref[...] = jax.nn.sigmoid(logit)


def _final(s2, y2, deg, batch, b2, wc1, bc1, wc2p, bc2r):
    return pl.pallas_call(
        _final_body,
        out_shape=jax.ShapeDtypeStruct((B, D), jnp.float32),
    )(s2, y2, deg, batch, b2, wc1, bc1, wc2p, bc2r)


def kernel(x0, edge_index0, batch0, x1, edge_index1, batch1,
           W1, b1, W2, b2, Wc1, bc1, Wc2, bc2):
    ei0 = jnp.stack([edge_index0[0].reshape(NT, NGRP, G2, K),
                     edge_index0[1].reshape(NT, NGRP, G2, K)], axis=2)
    ei1 = jnp.stack([edge_index1[0].reshape(NT, NGRP, G2, K),
                     edge_index1[1].reshape(NT, NGRP, G2, K)], axis=2)
    dst0 = ei0[:, :, 1]
    dst1 = ei1[:, :, 1]
    pad = NP - N
    xs = jnp.pad(jnp.stack([x0, x1]), ((0, 0), (0, pad), (0, 0)))

    ones128 = jnp.ones((K, D), jnp.float32)
    zeros128 = jnp.zeros((NPT, D), jnp.float32)

    deg0, deg1 = _deg_kernel(dst0, dst1, ones128, zeros128)
    deg = jnp.stack([deg0, deg1])

    y1 = _mm1(xs, deg, W1)
    s1a, s1b = _scatter_kernel(ei0, ei1, y1[0], y1[1], zeros128)
    s1 = jnp.stack([s1a, s1b])

    y2 = _mm2(s1, y1, deg, W2, b1.reshape(1, D))
    s2a, s2b = _scatter_kernel(ei0, ei1, y2[0], y2[1], zeros128)
    s2 = jnp.stack([s2a, s2b])

    batch = jnp.broadcast_to(
        jnp.pad(jnp.stack([batch0, batch1]), ((0, 0), (0, pad)),
                constant_values=B)[:, :, None], (2, NP, 16))
    wc2p = jnp.pad(Wc2, ((0, 0), (0, D - 1)))
    bc2r = jnp.broadcast_to(bc2[None, :], (1, D))
    out = _final(s2, y2, deg, batch, b2.reshape(1, D), Wc1,
                 bc1.reshape(1, D), wc2p, bc2r)
    return out[:, 0]


# speedup vs baseline: 23.3131x; 1.0515x over previous
import functools

import jax
import jax.numpy as jnp
from jax import lax
from jax.experimental import pallas as pl
from jax.experimental.pallas import tpu as pltpu
from jax.experimental.pallas import tpu_sc as plsc

N = 10000
NP = 10240
E = 320000
D = 128
B = 16

NT = 16
EPT = E // NT
NPT = NP // NT
K = 80
NCHUNK = EPT // K
G2 = 10
NGRP = NCHUNK // G2
NPAIR = NGRP // 2
NBUF = 4

_mesh = plsc.VectorSubcoreMesh(core_axis_name="c", subcore_axis_name="s")


@functools.partial(
    pl.kernel,
    out_type=(
        jax.ShapeDtypeStruct((NP, D), jnp.float32),
        jax.ShapeDtypeStruct((NP, D), jnp.float32),
    ),
    mesh=_mesh,
    scratch_types=[
        pltpu.VMEM((2, G2, K), jnp.int32),
        pltpu.VMEM((K, D), jnp.float32),
        pltpu.SemaphoreType.DMA,
        pltpu.VMEM_SHARED((NP, D), jnp.float32),
    ],
)
def _deg_kernel(dst0, dst1, ones_hbm, zeros_hbm, deg0, deg1,
                didx_v, ones_v, ssem, acc_sh):
    c = lax.axis_index("c")
    s = lax.axis_index("s")
    pltpu.sync_copy(zeros_hbm, acc_sh.at[pl.ds(s * NPT, NPT)])
    pltpu.sync_copy(ones_hbm, ones_v)
    plsc.subcore_barrier()

    def run(dst_hbm, out_hbm):
        def fire(rows):
            descs = [
                pltpu.async_copy(ones_v, acc_sh.at[r], ssem, add=True)
                for r in rows
            ]
            for d in descs:
                d.wait()

        def body(g, carry):
            pltpu.sync_copy(dst_hbm.at[s, pl.ds(2 * g, 2)], didx_v)
            fire([didx_v.at[p, t] for p in range(2) for t in range(G2)])
            return carry
        lax.fori_loop(0, NPAIR, body, 0)
        pltpu.sync_copy(dst_hbm.at[s, NGRP - 1], didx_v.at[0])
        fire([didx_v.at[0, t] for t in range(G2)])
        plsc.subcore_barrier()
        pltpu.sync_copy(acc_sh.at[pl.ds(s * NPT, NPT)],
                        out_hbm.at[pl.ds(s * NPT, NPT)])

    @pl.when(c == 0)
    def _():
        run(dst0, deg0)

    @pl.when(c == 1)
    def _():
        run(dst1, deg1)


@functools.partial(
    pl.kernel,
    out_type=(
        jax.ShapeDtypeStruct((NP, D), jnp.float32),
        jax.ShapeDtypeStruct((NP, D), jnp.float32),
    ),
    mesh=_mesh,
    scratch_types=[
        pltpu.VMEM((2, 2, G2, K), jnp.int32),
        [pltpu.VMEM((K, D), jnp.float32) for _ in range(NBUF)],
        pltpu.SemaphoreType.DMA,
        pltpu.SemaphoreType.DMA,
        pltpu.VMEM_SHARED((NP, D), jnp.float32),
    ],
)
def _scatter_kernel(ei0, ei1, y0, y1, zeros_hbm, s0_out, s1_out,
                    idx_v, rows_v, gsem, ssem, acc_sh):
    c = lax.axis_index("c")
    s = lax.axis_index("s")
    pltpu.sync_copy(zeros_hbm, acc_sh.at[pl.ds(s * NPT, NPT)])
    plsc.subcore_barrier()

    def run(ei_hbm, y_hbm, out_hbm):
        def pipe(srows, drows):
            n = len(srows)
            gds = [None] * n
            sds = [None] * n
            for t in range(n):
                if t >= NBUF:
                    sds[t - NBUF].wait()
                gds[t] = pltpu.async_copy(
                    y_hbm.at[srows[t]], rows_v[t % NBUF], gsem)
                if t >= NBUF - 1:
                    u = t - (NBUF - 1)
                    gds[u].wait()
                    sds[u] = pltpu.async_copy(
                        rows_v[u % NBUF], acc_sh.at[drows[u]],
                        ssem, add=True)
            for u in range(n - NBUF + 1, n):
                gds[u].wait()
                sds[u] = pltpu.async_copy(
                    rows_v[u % NBUF], acc_sh.at[drows[u]],
                    ssem, add=True)
            for u in range(n - NBUF, n):
                sds[u].wait()

        def body(g, carry):
            pltpu.sync_copy(ei_hbm.at[s, pl.ds(2 * g, 2)], idx_v)
            pipe([idx_v.at[p, 0, t] for p in range(2) for t in range(G2)],
                 [idx_v.at[p, 1, t] for p in range(2) for t in range(G2)])
            return carry
        lax.fori_loop(0, NPAIR, body, 0)
        pltpu.sync_copy(ei_hbm.at[s, NGRP - 1], idx_v.at[0])
        pipe([idx_v.at[0, 0, t] for t in range(G2)],
             [idx_v.at[0, 1, t] for t in range(G2)])
        plsc.subcore_barrier()
        pltpu.sync_copy(acc_sh.at[pl.ds(s * NPT, NPT)],
                        out_hbm.at[pl.ds(s * NPT, NPT)])

    @pl.when(c == 0)
    def _():
        run(ei0, y0, s0_out)

    @pl.when(c == 1)
    def _():
        run(ei1, y1, s1_out)


def _mm1_body(x_ref, deg_ref, w_ref, y_ref):
    dinv = lax.rsqrt(deg_ref[0][:, 0:1] + 1.0)
    y_ref[0] = dinv * jnp.dot(x_ref[0], w_ref[...],
                              preferred_element_type=jnp.float32)


NB = 4
RB = NP // NB


def _mm1(xs, deg, w1):
    return pl.pallas_call(
        _mm1_body,
        grid=(2, NB),
        in_specs=[
            pl.BlockSpec((1, RB, D), lambda g, r: (g, r, 0)),
            pl.BlockSpec((1, RB, D), lambda g, r: (g, r, 0)),
            pl.BlockSpec((D, D), lambda g, r: (0, 0)),
        ],
        out_specs=pl.BlockSpec((1, RB, D), lambda g, r: (g, r, 0)),
        out_shape=jax.ShapeDtypeStruct((2, NP, D), jnp.float32),
    )(xs, deg, w1)


def _mm2_body(s_ref, y_ref, deg_ref, w_ref, b_ref, out_ref):
    dinv = lax.rsqrt(deg_ref[0][:, 0:1] + 1.0)
    h = dinv * (s_ref[0] + y_ref[0]) + b_ref[...]
    out_ref[0] = dinv * jnp.dot(h, w_ref[...],
                                preferred_element_type=jnp.float32)


def _mm2(s1, y1, deg, w2, b1):
    return pl.pallas_call(
        _mm2_body,
        grid=(2, NB),
        in_specs=[
            pl.BlockSpec((1, RB, D), lambda g, r: (g, r, 0)),
            pl.BlockSpec((1, RB, D), lambda g, r: (g, r, 0)),
            pl.BlockSpec((1, RB, D), lambda g, r: (g, r, 0)),
            pl.BlockSpec((D, D), lambda g, r: (0, 0)),
            pl.BlockSpec((1, D), lambda g, r: (0, 0)),
        ],
        out_specs=pl.BlockSpec((1, RB, D), lambda g, r: (g, r, 0)),
        out_shape=jax.ShapeDtypeStruct((2, NP, D), jnp.float32),
    )(s1, y1, deg, w2, b1)


def _final_body(s_ref, y_ref, deg_ref, batch_ref, b2_ref, wc1_ref, bc1_ref,
                wc2_ref, bc2_ref, out_ref):
    feats = []
    for g in range(2):
        dinv = lax.rsqrt(deg_ref[g][:, 0:1] + 1.0)
        h2 = dinv * (s_ref[g] + y_ref[g]) + b2_ref[...]
        iota = lax.broadcasted_iota(jnp.int32, (NP, 16), 1)
        mask = (batch_ref[g] == iota).astype(jnp.float32)
        cnt = jnp.maximum(jnp.sum(mask, axis=0, keepdims=True), 1.0)
        meanmask = mask / cnt
        feats.append(lax.dot_general(
            meanmask, h2, (((0,), (0,)), ((), ())),
            preferred_element_type=jnp.float32))
    cf = jnp.concatenate(feats, axis=1)
    h = jax.nn.sigmoid(jnp.dot(cf, wc1_ref[...],
                               preferred_element_type=jnp.float32)
                       + bc1_ref[...])
    logit = jnp.dot(h, wc2_ref[...],
                    preferred_element_type=jnp.float32) + bc2_ref[...]
    out_ref[...] = jax.nn.sigmoid(logit)


def _final(s2, y2, deg, batch, b2, wc1, bc1, wc2p, bc2r):
    return pl.pallas_call(
        _final_body,
        out_shape=jax.ShapeDtypeStruct((B, D), jnp.float32),
    )(s2, y2, deg, batch, b2, wc1, bc1, wc2p, bc2r)


def kernel(x0, edge_index0, batch0, x1, edge_index1, batch1,
           W1, b1, W2, b2, Wc1, bc1, Wc2, bc2):
    ei0 = jnp.stack([edge_index0[0].reshape(NT, NGRP, G2, K),
                     edge_index0[1].reshape(NT, NGRP, G2, K)], axis=2)
    ei1 = jnp.stack([edge_index1[0].reshape(NT, NGRP, G2, K),
                     edge_index1[1].reshape(NT, NGRP, G2, K)], axis=2)
    dst0 = ei0[:, :, 1]
    dst1 = ei1[:, :, 1]
    pad = NP - N
    xs = jnp.pad(jnp.stack([x0, x1]), ((0, 0), (0, pad), (0, 0)))

    ones128 = jnp.ones((K, D), jnp.float32)
    zeros128 = jnp.zeros((NPT, D), jnp.float32)

    deg0, deg1 = _deg_kernel(dst0, dst1, ones128, zeros128)
    deg = jnp.stack([deg0, deg1])

    y1 = _mm1(xs, deg, W1)
    s1a, s1b = _scatter_kernel(ei0, ei1, y1[0], y1[1], zeros128)
    s1 = jnp.stack([s1a, s1b])

    y2 = _mm2(s1, y1, deg, W2, b1.reshape(1, D))
    s2a, s2b = _scatter_kernel(ei0, ei1, y2[0], y2[1], zeros128)
    s2 = jnp.stack([s2a, s2b])

    batch = jnp.broadcast_to(
        jnp.pad(jnp.stack([batch0, batch1]), ((0, 0), (0, pad)),
                constant_values=B)[:, :, None], (2, NP, 16))
    wc2p = jnp.pad(Wc2, ((0, 0), (0, D - 1)))
    bc2r = jnp.broadcast_to(bc2[None, :], (1, D))
    out = _final(s2, y2, deg, batch, b2.reshape(1, D), Wc1,
                 bc1.reshape(1, D), wc2p, bc2r)
    return out[:, 0]
